# baseline (speedup 1.0000x reference)
import functools

import jax
import jax.numpy as jnp
from jax import lax
from jax.experimental import pallas as pl
from jax.experimental.pallas import tpu as pltpu
from jax.experimental.pallas import tpu_sc as plsc

_NUM_CORES = 1
_NUM_WORKERS = 16 * _NUM_CORES
_RANK = 64
_LANES = 16
_CH = 128
_BN = 16384


def _pair(table):
    n = table.shape[0]
    return table[:n - n % 2].reshape(n // 2, 2 * table.shape[1])


def _repack(table):
    n, rank = table.shape
    m = ((n - 1) // 2 // _BN) * _BN
    rows_needed = max(m, (n - 1) - m)
    grid = (rows_needed + _BN - 1) // _BN
    rows = grid * _BN
    tt = table.T

    def body(t_ref1, t_ref2, out_ref):
        t1 = jnp.swapaxes(t_ref1[...], 0, 1)
        t2 = jnp.swapaxes(t_ref2[...], 0, 1)
        out_ref[...] = jnp.concatenate([t1, t2], axis=1)

    packed = pl.pallas_call(
        body,
        grid=(grid,),
        in_specs=[
            pl.BlockSpec((rank, _BN), lambda j: (0, j)),
            pl.BlockSpec((rank, _BN), functools.partial(
                lambda mb, j: (0, j + mb), m // _BN)),
        ],
        out_specs=pl.BlockSpec((_BN, 2 * rank), lambda j: (j, 0)),
        out_shape=jax.ShapeDtypeStruct((rows, 2 * rank), jnp.float32),
    )(tt, tt)
    return packed, m


def _make_sc_kernel(batch, nu_rows, mv, nv_rows):
    bpw = batch // _NUM_WORKERS
    nch = bpw // _CH

    mesh = plsc.VectorSubcoreMesh(
        core_axis_name="c", subcore_axis_name="s", num_cores=_NUM_CORES)

    @functools.partial(
        pl.kernel,
        out_type=jax.ShapeDtypeStruct((batch,), jnp.float32),
        mesh=mesh,
        compiler_params=pltpu.CompilerParams(use_tc_tiling_on_sc=True),
        scratch_types=(
            [
                pltpu.VMEM((bpw,), jnp.int32),
                pltpu.VMEM((bpw,), jnp.int32),
            ]
            + [pltpu.VMEM((_CH,), jnp.int32) for _ in range(2 * (bpw // _CH))]
            + [
                pltpu.VMEM((2, _CH, 2 * _RANK), jnp.float32),
                pltpu.VMEM((2, _CH, 2 * _RANK), jnp.float32),
                pltpu.VMEM((bpw,), jnp.float32),
                pltpu.SemaphoreType.DMA,
                pltpu.SemaphoreType.DMA,
            ]
        ),
    )
    def sc_kernel(users_hbm, items_hbm, up_hbm, vp_hbm, out_hbm,
                  uraw, iraw, *rest):
        pu = rest[:nch]
        pi = rest[nch:2 * nch]
        ubuf, vbuf, outv, sem0, sem1 = rest[2 * nch:]
        sems = [sem0, sem1]

        cid = lax.axis_index("c")
        sid = lax.axis_index("s")
        wid = sid * _NUM_CORES + cid
        base = wid * bpw

        pltpu.sync_copy(users_hbm.at[pl.ds(base, bpw)], uraw)
        pltpu.sync_copy(items_hbm.at[pl.ds(base, bpw)], iraw)

        for j in range(nch):
            for c in range(_CH // _LANES):
                off = j * _CH + c * _LANES
                uu = uraw[pl.ds(off, _LANES)]
                ii = iraw[pl.ds(off, _LANES)]
                pu[j][pl.ds(c * _LANES, _LANES)] = jnp.minimum(
                    lax.shift_right_logical(uu, 1), nu_rows - 1)
                pi[j][pl.ds(c * _LANES, _LANES)] = jnp.minimum(
                    jnp.where(ii >= mv, ii - mv, ii), nv_rows - 1)

        def fire(j):
            b = j % 2
            pltpu.async_copy(up_hbm.at[pu[j]], ubuf.at[b], sems[b])
            pltpu.async_copy(vp_hbm.at[pi[j]], vbuf.at[b], sems[b])

        def wait(j):
            b = j % 2
            pltpu.make_async_copy(up_hbm.at[pu[j]], ubuf.at[b], sems[b]).wait()
            pltpu.make_async_copy(vp_hbm.at[pi[j]], vbuf.at[b], sems[b]).wait()

        perms = [jnp.arange(_LANES, dtype=jnp.int32) ^ d for d in (8, 4, 2, 1)]
        onehot = [lax.iota(jnp.int32, _LANES) == i for i in range(_LANES)]

        def lane_sum(x):
            for p in perms:
                x = x + x.at[p].get(mode="promise_in_bounds")
            return x

        fire(0)
        for j in range(nch):
            if j + 1 < nch:
                fire(j + 1)
            wait(j)
            b = j % 2

            def body(g, _, j=j, b=b):
                res = jnp.zeros((_LANES,), jnp.float32)
                e0 = j * _CH + g * _LANES
                upar = (uraw[pl.ds(e0, _LANES)] & 1) * _RANK
                ipar = jnp.where(iraw[pl.ds(e0, _LANES)] >= mv, _RANK, 0)
                for i in range(_LANES):
                    r = g * _LANES + i
                    offu = upar[i]
                    offv = ipar[i]
                    acc = (ubuf[b, r, pl.ds(offu, _LANES)]
                           * vbuf[b, r, pl.ds(offv, _LANES)])
                    for c in range(1, _RANK // _LANES):
                        acc = acc + (
                            ubuf[b, r, pl.ds(offu + c * _LANES, _LANES)]
                            * vbuf[b, r, pl.ds(offv + c * _LANES, _LANES)])
                    res = jnp.where(onehot[i], lane_sum(acc), res)
                outv[pl.ds(j * _CH + g * _LANES, _LANES)] = res
                return 0

            lax.fori_loop(0, _CH // _LANES, body, 0)

        pltpu.sync_copy(outv, out_hbm.at[pl.ds(base, bpw)])

    return sc_kernel


def kernel(users, items, U, V):
    batch = users.shape[0]
    up = _pair(U)
    vp, mv = _repack(V)
    out = _make_sc_kernel(batch, up.shape[0], mv, vp.shape[0])(
        users.astype(jnp.int32), items.astype(jnp.int32), up, vp)
    return out


# speedup vs baseline: 1.0689x; 1.0689x over previous
import functools

import jax
import jax.numpy as jnp
from jax import lax
from jax.experimental import pallas as pl
from jax.experimental.pallas import tpu as pltpu
from jax.experimental.pallas import tpu_sc as plsc

_NUM_CORES = 1
_NUM_WORKERS = 16 * _NUM_CORES
_RANK = 64
_LANES = 16
_CH = 128
_BN = 16384


def _pair(table):
    n = table.shape[0]
    return table[:n - n % 2].reshape(n // 2, 2 * table.shape[1])


def _repack(table):
    n, rank = table.shape
    m = ((n - 1) // 2 // _BN) * _BN
    rows_needed = max(m, (n - 1) - m)
    grid = (rows_needed + _BN - 1) // _BN
    rows = grid * _BN
    tt = table.T

    def body(t_ref1, t_ref2, out_ref):
        t1 = jnp.swapaxes(t_ref1[...], 0, 1)
        t2 = jnp.swapaxes(t_ref2[...], 0, 1)
        out_ref[...] = jnp.concatenate([t1, t2], axis=1)

    packed = pl.pallas_call(
        body,
        grid=(grid,),
        in_specs=[
            pl.BlockSpec((rank, _BN), lambda j: (0, j)),
            pl.BlockSpec((rank, _BN), functools.partial(
                lambda mb, j: (0, j + mb), m // _BN)),
        ],
        out_specs=pl.BlockSpec((_BN, 2 * rank), lambda j: (j, 0)),
        out_shape=jax.ShapeDtypeStruct((rows, 2 * rank), jnp.float32),
    )(tt, tt)
    return packed, m


def _make_sc_kernel(batch, mu, nu_rows, mv, nv_rows):
    bpw = batch // _NUM_WORKERS
    nch = bpw // _CH

    mesh = plsc.VectorSubcoreMesh(
        core_axis_name="c", subcore_axis_name="s", num_cores=_NUM_CORES)

    @functools.partial(
        pl.kernel,
        out_type=jax.ShapeDtypeStruct((batch,), jnp.float32),
        mesh=mesh,
        compiler_params=pltpu.CompilerParams(use_tc_tiling_on_sc=True),
        scratch_types=(
            [
                pltpu.VMEM((bpw,), jnp.int32),
                pltpu.VMEM((bpw,), jnp.int32),
            ]
            + [pltpu.VMEM((_CH,), jnp.int32) for _ in range(2 * (bpw // _CH))]
            + [
                pltpu.VMEM((2, _CH, 2 * _RANK), jnp.float32),
                pltpu.VMEM((2, _CH, 2 * _RANK), jnp.float32),
                pltpu.VMEM((bpw,), jnp.float32),
                pltpu.SemaphoreType.DMA,
                pltpu.SemaphoreType.DMA,
            ]
        ),
    )
    def sc_kernel(users_hbm, items_hbm, up_hbm, vp_hbm, out_hbm,
                  uraw, iraw, *rest):
        pu = rest[:nch]
        pi = rest[nch:2 * nch]
        ubuf, vbuf, outv, sem0, sem1 = rest[2 * nch:]
        sems = [sem0, sem1]

        cid = lax.axis_index("c")
        sid = lax.axis_index("s")
        wid = sid * _NUM_CORES + cid
        base = wid * bpw

        pltpu.sync_copy(users_hbm.at[pl.ds(base, bpw)], uraw)
        pltpu.sync_copy(items_hbm.at[pl.ds(base, bpw)], iraw)

        for j in range(nch):
            for c in range(_CH // _LANES):
                off = j * _CH + c * _LANES
                uu = uraw[pl.ds(off, _LANES)]
                ii = iraw[pl.ds(off, _LANES)]
                pu[j][pl.ds(c * _LANES, _LANES)] = jnp.minimum(
                    jnp.where(uu >= mu, uu - mu, uu), nu_rows - 1)
                pi[j][pl.ds(c * _LANES, _LANES)] = jnp.minimum(
                    jnp.where(ii >= mv, ii - mv, ii), nv_rows - 1)

        def fire(j):
            b = j % 2
            pltpu.async_copy(up_hbm.at[pu[j]], ubuf.at[b], sems[b])
            pltpu.async_copy(vp_hbm.at[pi[j]], vbuf.at[b], sems[b])

        def wait(j):
            b = j % 2
            pltpu.make_async_copy(up_hbm.at[pu[j]], ubuf.at[b], sems[b]).wait()
            pltpu.make_async_copy(vp_hbm.at[pi[j]], vbuf.at[b], sems[b]).wait()

        perms = [jnp.arange(_LANES, dtype=jnp.int32) ^ d for d in (8, 4, 2, 1)]
        onehot = [lax.iota(jnp.int32, _LANES) == i for i in range(_LANES)]

        def lane_sum(x):
            for p in perms:
                x = x + x.at[p].get(mode="promise_in_bounds")
            return x

        fire(0)
        for j in range(nch):
            if j + 1 < nch:
                fire(j + 1)
            wait(j)
            b = j % 2

            def body(g, _, j=j, b=b):
                res = jnp.zeros((_LANES,), jnp.float32)
                e0 = j * _CH + g * _LANES
                upar = jnp.where(uraw[pl.ds(e0, _LANES)] >= mu, _RANK, 0)
                ipar = jnp.where(iraw[pl.ds(e0, _LANES)] >= mv, _RANK, 0)
                for i in range(_LANES):
                    r = g * _LANES + i
                    offu = upar[i]
                    offv = ipar[i]
                    acc = (ubuf[b, r, pl.ds(offu, _LANES)]
                           * vbuf[b, r, pl.ds(offv, _LANES)])
                    for c in range(1, _RANK // _LANES):
                        acc = acc + (
                            ubuf[b, r, pl.ds(offu + c * _LANES, _LANES)]
                            * vbuf[b, r, pl.ds(offv + c * _LANES, _LANES)])
                    res = jnp.where(onehot[i], lane_sum(acc), res)
                outv[pl.ds(j * _CH + g * _LANES, _LANES)] = res
                return 0

            lax.fori_loop(0, _CH // _LANES, body, 0)

        pltpu.sync_copy(outv, out_hbm.at[pl.ds(base, bpw)])

    return sc_kernel


def kernel(users, items, U, V):
    batch = users.shape[0]
    up, mu = _repack(U)
    vp, mv = _repack(V)
    out = _make_sc_kernel(batch, mu, up.shape[0], mv, vp.shape[0])(
        users.astype(jnp.int32), items.astype(jnp.int32), up, vp)
    return out
